# baseline (device time: 201465 ns/iter reference)
import jax
import jax.numpy as jnp
from jax import lax
from jax.experimental import pallas as pl
from jax.experimental.pallas import tpu as pltpu

N = 32
B, SQ, SKV, DM = 2, 512, 512, 768
HQ_PER, DH = 8, 64
FQ = HQ_PER * DH
ROWS = B * SQ
R = ROWS // N


def _body(x_ref, wq_ref, k_ref, v_ref, wo_ref, o_ref,
          q_s, ctx_s, p_s, scratch, send1, recv1, send2, recv2):
    me = lax.axis_index("i")

    bar = pltpu.get_barrier_semaphore()
    for k in range(1, N):
        j = lax.rem(me + k, N)
        pl.semaphore_signal(bar, inc=1, device_id=j,
                            device_id_type=pl.DeviceIdType.LOGICAL)

    xb = x_ref[:, :].astype(jnp.bfloat16)
    wqb = wq_ref[:, :].astype(jnp.bfloat16)
    q_s[:, :] = (jnp.dot(xb, wqb, preferred_element_type=jnp.float32)
                 * 0.125).astype(jnp.bfloat16)

    kb_i = lax.broadcasted_iota(jnp.int32, (1, SKV), 1) // 64

    for b in range(B):
        for h in range(HQ_PER):
            kht = k_ref[b, h * DH:(h + 1) * DH, :]
            vh = v_ref[b, :, h * DH:(h + 1) * DH]
            for qblk in range(SQ // 64):
                live = ((kb_i == qblk) | (kb_i == 0)
                        | ((qblk + kb_i) % 3 == 0))
                madd = jnp.where(live, 0.0, -1e9).astype(jnp.float32)
                r0 = b * SQ + qblk * 64
                qt = q_s[r0:r0 + 64, h * DH:(h + 1) * DH]
                s = jnp.dot(qt, kht,
                            preferred_element_type=jnp.float32)
                w = jnp.exp(s + madd)
                w = (w / jnp.sum(w, axis=-1, keepdims=True)).astype(jnp.bfloat16)
                ctx = jnp.dot(w, vh,
                              preferred_element_type=jnp.float32)
                ctx_s[r0:r0 + 64, h * DH:(h + 1) * DH] = (
                    ctx.astype(jnp.bfloat16))

    wob = wo_ref[:, :].astype(jnp.bfloat16)
    p_s[:, :] = jnp.dot(ctx_s[:, :], wob,
                        preferred_element_type=jnp.float32).astype(jnp.bfloat16)

    pl.semaphore_wait(bar, N - 1)

    sends = []

    for k in range(1, N):
        j = lax.rem(me + k, N)
        d = pltpu.make_async_remote_copy(
            src_ref=p_s.at[pl.ds(j * R, R), :],
            dst_ref=scratch.at[k - 1],
            send_sem=send1.at[k - 1],
            recv_sem=recv1.at[k - 1],
            device_id=j,
            device_id_type=pl.DeviceIdType.LOGICAL,
        )
        d.start()
        sends.append(d)

    acc = p_s[pl.ds(me * R, R), :].astype(jnp.float32)
    for k in range(1, N):
        w = pltpu.make_async_remote_copy(
            src_ref=p_s.at[pl.ds(0, R), :],
            dst_ref=scratch.at[k - 1],
            send_sem=send1.at[k - 1],
            recv_sem=recv1.at[k - 1],
            device_id=me,
            device_id_type=pl.DeviceIdType.LOGICAL,
        )
        w.wait_recv()
        acc = acc + scratch[k - 1].astype(jnp.float32)
    o_ref[pl.ds(me * R, R), :] = acc.astype(jnp.bfloat16)

    for k in range(1, N):
        j = lax.rem(me + k, N)
        d = pltpu.make_async_remote_copy(
            src_ref=o_ref.at[pl.ds(me * R, R), :],
            dst_ref=o_ref.at[pl.ds(me * R, R), :],
            send_sem=send2.at[k - 1],
            recv_sem=recv2.at[k - 1],
            device_id=j,
            device_id_type=pl.DeviceIdType.LOGICAL,
        )
        d.start()
        sends.append(d)

    for k in range(1, N):
        src_dev = lax.rem(me - k + N, N)
        w = pltpu.make_async_remote_copy(
            src_ref=o_ref.at[pl.ds(0, R), :],
            dst_ref=o_ref.at[pl.ds(src_dev * R, R), :],
            send_sem=send2.at[k - 1],
            recv_sem=recv2.at[k - 1],
            device_id=me,
            device_id_type=pl.DeviceIdType.LOGICAL,
        )
        w.wait_recv()

    for d in sends:
        d.wait_send()


def kernel(x, Wq, K_ext, V_ext, Wo):
    me = lax.axis_index("i")

    K2 = lax.dynamic_slice(
        K_ext.reshape(B, SKV, 256 * DH), (0, 0, me * FQ),
        (B, SKV, FQ)).astype(jnp.bfloat16)
    K2 = K2.transpose(0, 2, 1)
    V2 = lax.dynamic_slice(
        V_ext.reshape(B, SKV, 256 * DH), (0, 0, me * FQ),
        (B, SKV, FQ)).astype(jnp.bfloat16)
    x2 = x.reshape(ROWS, DM)

    out = pl.pallas_call(
        _body,
        out_shape=jax.ShapeDtypeStruct((ROWS, DM), jnp.bfloat16),
        in_specs=[pl.BlockSpec(memory_space=pltpu.VMEM)] * 5,
        out_specs=pl.BlockSpec(memory_space=pltpu.VMEM),
        scratch_shapes=[
            pltpu.VMEM((ROWS, FQ), jnp.bfloat16),
            pltpu.VMEM((ROWS, FQ), jnp.bfloat16),
            pltpu.VMEM((ROWS, DM), jnp.bfloat16),
            pltpu.VMEM((N - 1, R, DM), jnp.bfloat16),
            pltpu.SemaphoreType.DMA((N - 1,)),
            pltpu.SemaphoreType.DMA((N - 1,)),
            pltpu.SemaphoreType.DMA((N - 1,)),
            pltpu.SemaphoreType.DMA((N - 1,)),
        ],
        compiler_params=pltpu.CompilerParams(collective_id=0),
    )(x2, Wq, K2, V2, Wo)

    return out.reshape(B, SQ, DM).astype(jnp.float32)


# device time: 174723 ns/iter; 1.1531x vs baseline; 1.1531x over previous
import jax
import jax.numpy as jnp
from jax import lax
from jax.experimental import pallas as pl
from jax.experimental.pallas import tpu as pltpu

N = 32
B, SQ, SKV, DM = 2, 512, 512, 768
HQ_PER, DH = 8, 64
FQ = HQ_PER * DH
ROWS = B * SQ
R = ROWS // N


def _body(x_ref, wq_ref, k_ref, v_ref, wo_ref, o_ref,
          q_s, ctx_s, p_s, scratch, send1, recv1, send2, recv2):
    me = lax.axis_index("i")

    bar = pltpu.get_barrier_semaphore()
    for k in range(1, N):
        j = lax.rem(me + k, N)
        pl.semaphore_signal(bar, inc=1, device_id=j,
                            device_id_type=pl.DeviceIdType.LOGICAL)

    xb = x_ref[:, :].astype(jnp.bfloat16)
    wqb = wq_ref[:, :].astype(jnp.bfloat16)
    q_s[:, :] = (jnp.dot(xb, wqb, preferred_element_type=jnp.float32)
                 * 0.125).astype(jnp.bfloat16)

    qb = lax.broadcasted_iota(jnp.int32, (SQ, SKV), 0) // 64
    kb = lax.broadcasted_iota(jnp.int32, (SQ, SKV), 1) // 64
    live = (qb == kb) | (kb == 0) | ((qb + kb) % 3 == 0)
    madd = jnp.where(live, 0.0, -1e9).astype(jnp.float32)

    for b in range(B):
        for h in range(HQ_PER):
            qh = q_s[b * SQ:(b + 1) * SQ, h * DH:(h + 1) * DH]
            kht = k_ref[b, h * DH:(h + 1) * DH, :]
            vh = v_ref[b, :, h * DH:(h + 1) * DH]
            s = jnp.dot(qh, kht,
                        preferred_element_type=jnp.float32)
            w = jnp.exp(s + madd)
            w = w / jnp.sum(w, axis=-1, keepdims=True)
            ctx = jnp.dot(w.astype(jnp.bfloat16), vh,
                          preferred_element_type=jnp.float32)
            ctx_s[b * SQ:(b + 1) * SQ, h * DH:(h + 1) * DH] = (
                ctx.astype(jnp.bfloat16))

    wob = wo_ref[:, :].astype(jnp.bfloat16)
    p_s[:, :] = jnp.dot(ctx_s[:, :], wob,
                        preferred_element_type=jnp.float32).astype(jnp.bfloat16)

    pl.semaphore_wait(bar, N - 1)

    sends = []

    for k in range(1, N):
        j = lax.rem(me + k, N)
        d = pltpu.make_async_remote_copy(
            src_ref=p_s.at[pl.ds(j * R, R), :],
            dst_ref=scratch.at[k - 1],
            send_sem=send1.at[k - 1],
            recv_sem=recv1.at[k - 1],
            device_id=j,
            device_id_type=pl.DeviceIdType.LOGICAL,
        )
        d.start()
        sends.append(d)

    acc = p_s[pl.ds(me * R, R), :].astype(jnp.float32)
    for k in range(1, N):
        w = pltpu.make_async_remote_copy(
            src_ref=p_s.at[pl.ds(0, R), :],
            dst_ref=scratch.at[k - 1],
            send_sem=send1.at[k - 1],
            recv_sem=recv1.at[k - 1],
            device_id=me,
            device_id_type=pl.DeviceIdType.LOGICAL,
        )
        w.wait_recv()
        acc = acc + scratch[k - 1].astype(jnp.float32)
    o_ref[pl.ds(me * R, R), :] = acc.astype(jnp.bfloat16)

    for k in range(1, N):
        j = lax.rem(me + k, N)
        d = pltpu.make_async_remote_copy(
            src_ref=o_ref.at[pl.ds(me * R, R), :],
            dst_ref=o_ref.at[pl.ds(me * R, R), :],
            send_sem=send2.at[k - 1],
            recv_sem=recv2.at[k - 1],
            device_id=j,
            device_id_type=pl.DeviceIdType.LOGICAL,
        )
        d.start()
        sends.append(d)

    for k in range(1, N):
        src_dev = lax.rem(me - k + N, N)
        w = pltpu.make_async_remote_copy(
            src_ref=o_ref.at[pl.ds(0, R), :],
            dst_ref=o_ref.at[pl.ds(src_dev * R, R), :],
            send_sem=send2.at[k - 1],
            recv_sem=recv2.at[k - 1],
            device_id=me,
            device_id_type=pl.DeviceIdType.LOGICAL,
        )
        w.wait_recv()

    for d in sends:
        d.wait_send()


def kernel(x, Wq, K_ext, V_ext, Wo):
    me = lax.axis_index("i")

    K2 = lax.dynamic_slice(
        K_ext.reshape(B, SKV, 256 * DH), (0, 0, me * FQ),
        (B, SKV, FQ)).astype(jnp.bfloat16)
    K2 = K2.transpose(0, 2, 1)
    V2 = lax.dynamic_slice(
        V_ext.reshape(B, SKV, 256 * DH), (0, 0, me * FQ),
        (B, SKV, FQ)).astype(jnp.bfloat16)
    x2 = x.reshape(ROWS, DM)

    out = pl.pallas_call(
        _body,
        out_shape=jax.ShapeDtypeStruct((ROWS, DM), jnp.bfloat16),
        in_specs=[pl.BlockSpec(memory_space=pltpu.VMEM)] * 5,
        out_specs=pl.BlockSpec(memory_space=pltpu.VMEM),
        scratch_shapes=[
            pltpu.VMEM((ROWS, FQ), jnp.bfloat16),
            pltpu.VMEM((ROWS, FQ), jnp.bfloat16),
            pltpu.VMEM((ROWS, DM), jnp.bfloat16),
            pltpu.VMEM((N - 1, R, DM), jnp.bfloat16),
            pltpu.SemaphoreType.DMA((N - 1,)),
            pltpu.SemaphoreType.DMA((N - 1,)),
            pltpu.SemaphoreType.DMA((N - 1,)),
            pltpu.SemaphoreType.DMA((N - 1,)),
        ],
        compiler_params=pltpu.CompilerParams(collective_id=0),
    )(x2, Wq, K2, V2, Wo)

    return out.reshape(B, SQ, DM).astype(jnp.float32)
